# Initial kernel scaffold; baseline (speedup 1.0000x reference)
import jax
import jax.numpy as jnp
from jax.experimental import pallas as pl


def kernel(xyz, tables, W0, W1, W2):
    raise NotImplementedError("write your pallas kernel here")


# speedup vs baseline: 7.9798x; 7.9798x over previous
import functools

import numpy as np
import jax
import jax.numpy as jnp
from jax import lax
from jax.experimental import pallas as pl
from jax.experimental.pallas import tpu as pltpu
from jax.experimental.pallas import tpu_sc as plsc

_N_LEVELS = 16
_F = 2
_T = 1 << 19
_BASE_RES = 16
_SCALE = 1.5
_H = 512
_W_IMG = 512
_N = _H * _W_IMG
_D_IN = _N_LEVELS * _F
_PRIME = int(np.uint32(2654435761).astype(np.int32))

_LEVELS = []
for _l in range(_N_LEVELS):
    _res = int(np.floor(_BASE_RES * (_SCALE ** _l)))
    _stride = _res + 1
    _size = min(_T, _stride * _stride)
    _LEVELS.append((_res, _stride, _size, _stride * _stride <= _size))

_NC = 2
_NS = 16
_NW = _NC * _NS
_PPW = _N // _NW
_C = 512
_NCHUNK = _PPW // _C
_NBLK = _N // _C


def _sc_encode(x, y, tx, ty):
    mesh = plsc.VectorSubcoreMesh(core_axis_name="c", subcore_axis_name="s")
    scratch = [
        pltpu.VMEM((_C,), jnp.float32),
        pltpu.VMEM((_C,), jnp.float32),
        pltpu.VMEM((_C,), jnp.float32),
        pltpu.VMEM((_C,), jnp.float32),
        pltpu.VMEM((_C,), jnp.int32),
        pltpu.VMEM((_C,), jnp.int32),
        pltpu.VMEM((_C,), jnp.int32),
        pltpu.VMEM((_C,), jnp.int32),
        pltpu.VMEM((_C,), jnp.float32),
        pltpu.VMEM((_C,), jnp.float32),
        pltpu.VMEM((_C,), jnp.float32),
        pltpu.VMEM((_C,), jnp.float32),
        pltpu.VMEM((_C,), jnp.float32),
        pltpu.VMEM((_C,), jnp.float32),
        pltpu.VMEM((_C,), jnp.float32),
        pltpu.VMEM((_C,), jnp.float32),
        pltpu.VMEM((_D_IN * _C,), jnp.float32),
        pltpu.SemaphoreType.DMA,
    ]

    @functools.partial(
        pl.kernel,
        out_type=jax.ShapeDtypeStruct((_NBLK, _D_IN * _C), jnp.float32),
        mesh=mesh,
        scratch_types=scratch,
    )
    def k(x_hbm, y_hbm, *rest):
        tx_hbm = rest[:_N_LEVELS]
        ty_hbm = rest[_N_LEVELS:2 * _N_LEVELS]
        enc_hbm = rest[2 * _N_LEVELS]
        (x_v, y_v, wx_v, wy_v, i00_v, i01_v, i10_v, i11_v,
         f00x_v, f01x_v, f10x_v, f11x_v,
         f00y_v, f01y_v, f10y_v, f11y_v, enc_v, sem) = rest[2 * _N_LEVELS + 1:]

        wid = lax.axis_index("s") * _NC + lax.axis_index("c")
        lane = lax.iota(jnp.int32, 16)

        def chunk_body(ci, carry):
            base = wid * _PPW + ci * _C
            pltpu.sync_copy(x_hbm.at[pl.ds(base, _C)], x_v)
            pltpu.sync_copy(y_hbm.at[pl.ds(base, _C)], y_v)

            for l in range(_N_LEVELS):
                res, stride, size, dense = _LEVELS[l]

                def body_a(i, c, res=res, stride=stride, dense=dense):
                    s = pl.ds(i * 16, 16)
                    px = x_v[s] * float(res)
                    py = y_v[s] * float(res)
                    ix0 = px.astype(jnp.int32)
                    iy0 = py.astype(jnp.int32)
                    wx_v[s] = px - ix0.astype(jnp.float32)
                    wy_v[s] = py - iy0.astype(jnp.float32)
                    ix1 = ix0 + 1
                    iy1 = iy0 + 1
                    if dense:
                        r0 = ix0 * stride
                        r1 = ix1 * stride
                        i00_v[s] = r0 + iy0
                        i01_v[s] = r0 + iy1
                        i10_v[s] = r1 + iy0
                        i11_v[s] = r1 + iy1
                    else:
                        m = _T - 1
                        h0 = iy0 * _PRIME
                        h1 = iy1 * _PRIME
                        i00_v[s] = (ix0 ^ h0) & m
                        i01_v[s] = (ix0 ^ h1) & m
                        i10_v[s] = (ix1 ^ h0) & m
                        i11_v[s] = (ix1 ^ h1) & m
                    return c

                lax.fori_loop(0, _C // 16, body_a, 0)

                cps = [
                    pltpu.async_copy(tx_hbm[l].at[i00_v], f00x_v, sem),
                    pltpu.async_copy(tx_hbm[l].at[i01_v], f01x_v, sem),
                    pltpu.async_copy(tx_hbm[l].at[i10_v], f10x_v, sem),
                    pltpu.async_copy(tx_hbm[l].at[i11_v], f11x_v, sem),
                    pltpu.async_copy(ty_hbm[l].at[i00_v], f00y_v, sem),
                    pltpu.async_copy(ty_hbm[l].at[i01_v], f01y_v, sem),
                    pltpu.async_copy(ty_hbm[l].at[i10_v], f10y_v, sem),
                    pltpu.async_copy(ty_hbm[l].at[i11_v], f11y_v, sem),
                ]
                for cp in cps:
                    cp.wait()

                off0 = (2 * l) * _C
                off1 = (2 * l + 1) * _C

                def body_b(i, c, off0=off0, off1=off1):
                    s = pl.ds(i * 16, 16)
                    wx = wx_v[s]
                    wy = wy_v[s]
                    u = 1.0 - wx
                    v = 1.0 - wy
                    w00 = u * v
                    w01 = u * wy
                    w10 = wx * v
                    w11 = wx * wy
                    enc_v[pl.ds(off0 + i * 16, 16)] = (
                        f00x_v[s] * w00 + f01x_v[s] * w01
                        + f10x_v[s] * w10 + f11x_v[s] * w11)
                    enc_v[pl.ds(off1 + i * 16, 16)] = (
                        f00y_v[s] * w00 + f01y_v[s] * w01
                        + f10y_v[s] * w10 + f11y_v[s] * w11)
                    return c

                lax.fori_loop(0, _C // 16, body_b, 0)

            blk = wid * _NCHUNK + ci
            pltpu.sync_copy(enc_v, enc_hbm.at[blk])
            return carry

        lax.fori_loop(0, _NCHUNK, chunk_body, 0)

    return k(x, y, *tx, *ty)


def _tc_mlp(enc, W0, W1, W2):
    bm = 2048

    def body(e_ref, w0_ref, w1_ref, w2_ref, o_ref):
        h = jnp.dot(e_ref[...], w0_ref[...], preferred_element_type=jnp.float32)
        h = jnp.maximum(h, 0.0)
        h = jnp.dot(h, w1_ref[...], preferred_element_type=jnp.float32)
        h = jnp.maximum(h, 0.0)
        o = jnp.dot(h, w2_ref[...], preferred_element_type=jnp.float32)
        o_ref[...] = jnp.clip(o, 0.0, 1.0)

    return pl.pallas_call(
        body,
        grid=(_N // bm,),
        in_specs=[
            pl.BlockSpec((bm, _D_IN), lambda i: (i, 0)),
            pl.BlockSpec((_D_IN, 64), lambda i: (0, 0)),
            pl.BlockSpec((64, 64), lambda i: (0, 0)),
            pl.BlockSpec((64, 2), lambda i: (0, 0)),
        ],
        out_specs=pl.BlockSpec((bm, 2), lambda i: (i, 0)),
        out_shape=jax.ShapeDtypeStruct((_N, 2), jnp.float32),
    )(enc, W0, W1, W2)


def kernel(xyz, tables, W0, W1, W2):
    x = xyz[:, 0]
    y = xyz[:, 1]
    tx = [t[:, 0] for t in tables]
    ty = [t[:, 1] for t in tables]
    enc_blk = _sc_encode(x, y, tx, ty)
    enc = enc_blk.reshape(_NBLK, _D_IN, _C).transpose(0, 2, 1).reshape(_N, _D_IN)
    out = _tc_mlp(enc, W0, W1, W2)
    out = out.reshape(_H, _W_IMG, 2)
    return lax.complex(out[..., 0], out[..., 1])[None, None]
